# Initial kernel scaffold; baseline (speedup 1.0000x reference)
import jax
import jax.numpy as jnp
from jax.experimental import pallas as pl


def kernel(x, table):
    raise NotImplementedError("write your pallas kernel here")


# speedup vs baseline: 1.4801x; 1.4801x over previous
import functools

import jax
import jax.numpy as jnp
from jax import lax
from jax.experimental import pallas as pl
from jax.experimental.pallas import tpu as pltpu
from jax.experimental.pallas import tpu_sc as plsc

_INFO = plsc.get_sparse_core_info()
_NC = _INFO.num_cores
_NS = _INFO.num_subcores
_NW = _NC * _NS

_CHUNK = 64
_NBUF = 2


def _build_gather(B, V, D):
    assert B % _NW == 0
    b_per_w = B // _NW
    assert b_per_w % _CHUNK == 0
    n_chunks = b_per_w // _CHUNK

    mesh = plsc.VectorSubcoreMesh(core_axis_name="c", subcore_axis_name="s")

    @functools.partial(
        pl.kernel,
        mesh=mesh,
        out_type=jax.ShapeDtypeStruct((B, D), jnp.float32),
        scratch_types=[
            pltpu.VMEM((n_chunks, _CHUNK), jnp.int32),
            pltpu.VMEM((_NBUF, _CHUNK, D), jnp.float32),
            pltpu.SemaphoreType.DMA,
            pltpu.SemaphoreType.DMA,
        ],
    )
    def k(idx_hbm, table_hbm, out_hbm, idx_v, rows_v, sem0, sem1):
        wid = lax.axis_index("s") * _NC + lax.axis_index("c")
        base = wid * b_per_w
        pltpu.sync_copy(idx_hbm.at[wid], idx_v)
        sems = [sem0, sem1]
        cps = [None] * _NBUF
        for c in range(n_chunks):
            b = c % _NBUF
            cps[b] = pltpu.async_copy(
                table_hbm.at[idx_v.at[c]], rows_v.at[b], sems[b]
            )
            if c >= 1:
                pb = (c - 1) % _NBUF
                cps[pb].wait()
                pltpu.sync_copy(
                    rows_v.at[pb],
                    out_hbm.at[pl.ds(base + (c - 1) * _CHUNK, _CHUNK)],
                )
        lb = (n_chunks - 1) % _NBUF
        cps[lb].wait()
        pltpu.sync_copy(
            rows_v.at[lb],
            out_hbm.at[pl.ds(base + (n_chunks - 1) * _CHUNK, _CHUNK)],
        )

    return k


def kernel(x, table):
    B = x.size
    V, D = table.shape
    b_per_w = B // _NW
    idx = x.reshape(_NW, b_per_w // _CHUNK, _CHUNK).astype(jnp.int32)
    out = _build_gather(B, V, D)(idx, table)
    return out.reshape(*x.shape, D)
